# baseline (speedup 1.0000x reference)
import jax
import jax.numpy as jnp
from jax import lax
from jax.experimental import pallas as pl
from jax.experimental.pallas import tpu as pltpu
from jax.experimental.pallas import tpu_sc as plsc

NUM_CLASS = 1000
MEM_SIZE = 50
SLOT_PAD = 56
D = 512
NQ = 64
NV = 4096

ROWS_PAD = NUM_CLASS * SLOT_PAD + 64
DUMMY_ROW = NUM_CLASS * SLOT_PAD

NC, NS = 2, 16
NW = NC * NS
VPW = NV // NW

CB = 40
IRB = CB * MEM_SIZE
ORB = CB * SLOT_PAD

_PREC = jax.lax.Precision.DEFAULT


def _stage_body(x_ref, o_ref):
    x = x_ref[...]
    n = jnp.sqrt(jnp.sum(x * x, axis=1, keepdims=True))
    xn = x * (1.0 / (n + 1e-8))
    for i in range(CB):
        o_ref[SLOT_PAD * i:SLOT_PAD * i + MEM_SIZE, :] = (
            xn[MEM_SIZE * i:MEM_SIZE * (i + 1), :])
        o_ref[SLOT_PAD * i + MEM_SIZE:SLOT_PAD * (i + 1), :] = (
            jnp.zeros((SLOT_PAD - MEM_SIZE, D), jnp.float32))


def _stage(mem):
    return pl.pallas_call(
        _stage_body,
        grid=(NUM_CLASS // CB,),
        in_specs=[pl.BlockSpec((IRB, D), lambda i: (i, 0))],
        out_specs=pl.BlockSpec((ORB, D), lambda i: (i, 0)),
        out_shape=jax.ShapeDtypeStruct((ROWS_PAD, D), jnp.float32),
    )(mem)


def _winner_body(icol_ref, irow_ref, o_ref):
    i = pl.program_id(0)
    a = icol_ref[...]
    b = irow_ref[...]
    jc = 512 * i + lax.broadcasted_iota(jnp.int32, (512, 1), 0)
    jr = lax.broadcasted_iota(jnp.int32, (1, NV), 1)
    dead = jnp.any((a == b) & (jr > jc), axis=1, keepdims=True)
    c = a // MEM_SIZE
    padded = SLOT_PAD * c + (a - MEM_SIZE * c)
    o_ref[...] = jnp.where(dead, DUMMY_ROW, padded)


def _winner(idx):
    icol = idx.reshape(NV, 1)
    irow = idx.reshape(1, NV)
    out = pl.pallas_call(
        _winner_body,
        grid=(NV // 512,),
        in_specs=[
            pl.BlockSpec((512, 1), lambda i: (i, 0)),
            pl.BlockSpec((1, NV), lambda i: (0, 0)),
        ],
        out_specs=pl.BlockSpec((512, 1), lambda i: (i, 0)),
        out_shape=jax.ShapeDtypeStruct((NV, 1), jnp.int32),
    )(icol, irow)
    return out.reshape(NW, 2, NV // NW // 2)


def _norm_body(x_ref, o_ref):
    x = x_ref[...]
    n = jnp.sqrt(jnp.sum(x * x, axis=1, keepdims=True))
    o_ref[...] = x * (1.0 / (n + 1e-8))


def _valn(val):
    return pl.pallas_call(
        _norm_body,
        grid=(NV // 2048,),
        in_specs=[pl.BlockSpec((2048, D), lambda i: (i, 0))],
        out_specs=pl.BlockSpec((2048, D), lambda i: (i, 0)),
        out_shape=jax.ShapeDtypeStruct((NV, D), jnp.float32),
    )(val)


def _qn(queries):
    return pl.pallas_call(
        _norm_body,
        out_shape=jax.ShapeDtypeStruct((NQ, D), jnp.float32),
    )(queries)


HPW = VPW // 2


def _scatter_body(mem_ref, valn_ref, idx_ref, idx_a, idx_b, rows_a, rows_b,
                  sem_a, sem_b, sem_sa, sem_sb):
    wid = lax.axis_index("s") * NC + lax.axis_index("c")
    base = wid * VPW
    pltpu.sync_copy(idx_ref.at[wid, 0], idx_a)
    pltpu.sync_copy(idx_ref.at[wid, 1], idx_b)
    ca = pltpu.async_copy(valn_ref.at[pl.ds(base, HPW)], rows_a, sem_a)
    cb = pltpu.async_copy(valn_ref.at[pl.ds(base + HPW, HPW)], rows_b, sem_b)
    ca.wait()
    sa = pltpu.async_copy(rows_a, mem_ref.at[idx_a], sem_sa)
    cb.wait()
    sb = pltpu.async_copy(rows_b, mem_ref.at[idx_b], sem_sb)
    sa.wait()
    sb.wait()


def _scatter(mem_ref, valn, idx2):
    k = pl.kernel(
        _scatter_body,
        out_type=(),
        mesh=plsc.VectorSubcoreMesh(core_axis_name="c", subcore_axis_name="s"),
        scratch_types=[
            pltpu.VMEM((HPW,), jnp.int32),
            pltpu.VMEM((HPW,), jnp.int32),
            pltpu.VMEM((HPW, D), jnp.float32),
            pltpu.VMEM((HPW, D), jnp.float32),
            pltpu.SemaphoreType.DMA,
            pltpu.SemaphoreType.DMA,
            pltpu.SemaphoreType.DMA,
            pltpu.SemaphoreType.DMA,
        ],
    )
    k(mem_ref, valn, idx2)


def _readout_body(x_ref, q_ref, o_ref):
    qn = q_ref[...]
    x = x_ref[...]
    st = lax.dot_general(x, qn, (((1,), (1,)), ((), ())),
                         precision=_PREC)
    s3 = st.reshape(CB, SLOT_PAD, NQ)
    live = lax.broadcasted_iota(jnp.int32, (1, SLOT_PAD, 1), 1) < MEM_SIZE
    sims = jnp.where(live, 100.0 * s3, -3.0e38)
    mx = jnp.max(sims, axis=1, keepdims=True)
    e = jnp.exp(sims - mx)
    w3 = e * (1.0 / jnp.sum(e, axis=1, keepdims=True))
    num = jnp.sum(w3 * s3, axis=1)
    xn3 = x.reshape(CB, SLOT_PAD, D)
    rows = []
    for i in range(CB):
        p = lax.dot_general(w3[i], xn3[i], (((0,), (0,)), ((), ())),
                            precision=_PREC)
        pn = jnp.sqrt(jnp.sum(p * p, axis=1))
        rows.append((100.0 * num[i] / (pn + 1e-8))[None, :])
    o_ref[...] = jnp.concatenate(rows, axis=0)


def _readout(memn, qn):
    return pl.pallas_call(
        _readout_body,
        grid=(NUM_CLASS // CB,),
        in_specs=[
            pl.BlockSpec((ORB, D), lambda i: (i, 0)),
            pl.BlockSpec((NQ, D), lambda i: (0, 0)),
        ],
        out_specs=pl.BlockSpec((CB, NQ), lambda i: (i, 0)),
        out_shape=jax.ShapeDtypeStruct((NUM_CLASS, NQ), jnp.float32),
    )(memn, qn)


def _transpose_body(x_ref, o_ref):
    o_ref[...] = x_ref[...].T


def _transpose(lt):
    return pl.pallas_call(
        _transpose_body,
        out_shape=jax.ShapeDtypeStruct((NQ, NUM_CLASS), jnp.float32),
    )(lt)


def kernel(mem, val, queries, idx):
    memn = _stage(mem)
    idx2 = _winner(idx)
    valn = _valn(val)
    qn = _qn(queries)
    mem_ref = jax.new_ref(memn)
    _scatter(mem_ref, valn, idx2)
    return _transpose(_readout(mem_ref[...], qn))


# speedup vs baseline: 1.0048x; 1.0048x over previous
import jax
import jax.numpy as jnp
from jax import lax
from jax.experimental import pallas as pl
from jax.experimental.pallas import tpu as pltpu
from jax.experimental.pallas import tpu_sc as plsc

NUM_CLASS = 1000
MEM_SIZE = 50
SLOT_PAD = 56
D = 512
NQ = 64
NV = 4096

ROWS_PAD = NUM_CLASS * SLOT_PAD + 64
DUMMY_ROW = NUM_CLASS * SLOT_PAD

NC, NS = 2, 16
NW = NC * NS
VPW = NV // NW

CB = 40
IRB = CB * MEM_SIZE
ORB = CB * SLOT_PAD

_PREC = jax.lax.Precision.DEFAULT


def _stage_body(x_ref, o_ref):
    x = x_ref[...]
    n = jnp.sqrt(jnp.sum(x * x, axis=1, keepdims=True))
    xn = x * (1.0 / (n + 1e-8))
    for i in range(CB):
        o_ref[SLOT_PAD * i:SLOT_PAD * i + MEM_SIZE, :] = (
            xn[MEM_SIZE * i:MEM_SIZE * (i + 1), :])
        o_ref[SLOT_PAD * i + MEM_SIZE:SLOT_PAD * (i + 1), :] = (
            jnp.zeros((SLOT_PAD - MEM_SIZE, D), jnp.float32))


def _stage(mem):
    return pl.pallas_call(
        _stage_body,
        grid=(NUM_CLASS // CB,),
        in_specs=[pl.BlockSpec((IRB, D), lambda i: (i, 0))],
        out_specs=pl.BlockSpec((ORB, D), lambda i: (i, 0)),
        out_shape=jax.ShapeDtypeStruct((ROWS_PAD, D), jnp.float32),
    )(mem)


def _winner_body(icol_ref, irow_ref, o_ref):
    i = pl.program_id(0)
    a = icol_ref[...]
    b = irow_ref[...]
    jc = 512 * i + lax.broadcasted_iota(jnp.int32, (512, 1), 0)
    jr = lax.broadcasted_iota(jnp.int32, (1, NV), 1)
    dead = jnp.any((a == b) & (jr > jc), axis=1, keepdims=True)
    c = a // MEM_SIZE
    padded = SLOT_PAD * c + (a - MEM_SIZE * c)
    o_ref[...] = jnp.where(dead, DUMMY_ROW, padded)


def _winner(idx):
    icol = idx.reshape(NV, 1)
    irow = idx.reshape(1, NV)
    out = pl.pallas_call(
        _winner_body,
        grid=(NV // 512,),
        in_specs=[
            pl.BlockSpec((512, 1), lambda i: (i, 0)),
            pl.BlockSpec((1, NV), lambda i: (0, 0)),
        ],
        out_specs=pl.BlockSpec((512, 1), lambda i: (i, 0)),
        out_shape=jax.ShapeDtypeStruct((NV, 1), jnp.int32),
    )(icol, irow)
    return out.reshape(NW, VPW)


def _norm_body(x_ref, o_ref):
    x = x_ref[...]
    n = jnp.sqrt(jnp.sum(x * x, axis=1, keepdims=True))
    o_ref[...] = x * (1.0 / (n + 1e-8))


def _valn(val):
    return pl.pallas_call(
        _norm_body,
        grid=(NV // 2048,),
        in_specs=[pl.BlockSpec((2048, D), lambda i: (i, 0))],
        out_specs=pl.BlockSpec((2048, D), lambda i: (i, 0)),
        out_shape=jax.ShapeDtypeStruct((NV, D), jnp.float32),
    )(val)


def _qn(queries):
    return pl.pallas_call(
        _norm_body,
        out_shape=jax.ShapeDtypeStruct((NQ, D), jnp.float32),
    )(queries)


def _scatter_body(mem_ref, valn_ref, idx_ref, idx_v, rows_v, sem):
    wid = lax.axis_index("s") * NC + lax.axis_index("c")
    pltpu.sync_copy(idx_ref.at[wid], idx_v)
    pltpu.sync_copy(valn_ref.at[pl.ds(wid * VPW, VPW)], rows_v)
    pltpu.async_copy(rows_v, mem_ref.at[idx_v], sem).wait()


def _scatter(mem_ref, valn, idx2):
    k = pl.kernel(
        _scatter_body,
        out_type=(),
        mesh=plsc.VectorSubcoreMesh(core_axis_name="c", subcore_axis_name="s"),
        scratch_types=[
            pltpu.VMEM((VPW,), jnp.int32),
            pltpu.VMEM((VPW, D), jnp.float32),
            pltpu.SemaphoreType.DMA,
        ],
    )
    k(mem_ref, valn, idx2)


def _readout_body(x_ref, q_ref, o_ref):
    qn = q_ref[...]
    x = x_ref[...]
    st = lax.dot_general(x, qn, (((1,), (1,)), ((), ())),
                         precision=_PREC)
    s3 = st.reshape(CB, SLOT_PAD, NQ)
    live = lax.broadcasted_iota(jnp.int32, (1, SLOT_PAD, 1), 1) < MEM_SIZE
    sims = jnp.where(live, 100.0 * s3, -3.0e38)
    mx = jnp.max(sims, axis=1, keepdims=True)
    e = jnp.exp(sims - mx)
    w3 = e * (1.0 / jnp.sum(e, axis=1, keepdims=True))
    num = jnp.sum(w3 * s3, axis=1)
    xn3 = x.reshape(CB, SLOT_PAD, D)
    rows = []
    for i in range(CB):
        p = lax.dot_general(w3[i], xn3[i], (((0,), (0,)), ((), ())),
                            precision=_PREC)
        pn = jnp.sqrt(jnp.sum(p * p, axis=1))
        rows.append((100.0 * num[i] / (pn + 1e-8))[None, :])
    o_ref[...] = jnp.concatenate(rows, axis=0)


def _readout(memn, qn):
    return pl.pallas_call(
        _readout_body,
        grid=(NUM_CLASS // CB,),
        in_specs=[
            pl.BlockSpec((ORB, D), lambda i: (i, 0)),
            pl.BlockSpec((NQ, D), lambda i: (0, 0)),
        ],
        out_specs=pl.BlockSpec((CB, NQ), lambda i: (i, 0)),
        out_shape=jax.ShapeDtypeStruct((NUM_CLASS, NQ), jnp.float32),
    )(memn, qn)


def _transpose_body(x_ref, o_ref):
    o_ref[...] = x_ref[...].T


def _transpose(lt):
    return pl.pallas_call(
        _transpose_body,
        out_shape=jax.ShapeDtypeStruct((NQ, NUM_CLASS), jnp.float32),
    )(lt)


def kernel(mem, val, queries, idx):
    memn = _stage(mem)
    idx2 = _winner(idx)
    valn = _valn(val)
    qn = _qn(queries)
    mem_ref = jax.new_ref(memn)
    _scatter(mem_ref, valn, idx2)
    return _transpose(_readout(mem_ref[...], qn))
